# Initial kernel scaffold; baseline (speedup 1.0000x reference)
import jax
import jax.numpy as jnp
from jax.experimental import pallas as pl


def kernel(x, table, W, b):
    raise NotImplementedError("write your pallas kernel here")


# speedup vs baseline: 2.4316x; 2.4316x over previous
import functools

import jax
import jax.numpy as jnp
from jax import lax
from jax.experimental import pallas as pl
from jax.experimental.pallas import tpu as pltpu
from jax.experimental.pallas import tpu_sc as plsc

B = 4096
L = 200
D = 32
V = 1000000

NC = 2
NS = 16
NW = NC * NS
RW = B // NW
CH = 100
CPR = L // CH
NBUF = 4
UNROLL = 8

_mesh = plsc.VectorSubcoreMesh(
    core_axis_name="c", subcore_axis_name="s", num_cores=NC, num_subcores=NS)


def _body(x_hbm, tab_hbm, par_hbm, out_hbm, idx_v, rows_v, par_v, out_v,
          *sems):
  c = lax.axis_index("c")
  s = lax.axis_index("s")
  w = s * NC + c

  pltpu.sync_copy(x_hbm.at[pl.ds(w * (RW * CPR), RW * CPR)], idx_v)
  pltpu.sync_copy(par_hbm, par_v)

  def fire(buf, r):
    for h in range(CPR):
      pltpu.async_copy(
          tab_hbm.at[idx_v.at[CPR * r + h]],
          rows_v.at[buf, pl.ds(h * CH, CH)],
          sems[buf])

  def drain(buf):
    pltpu.make_async_copy(
        tab_hbm.at[pl.ds(0, L)], rows_v.at[buf], sems[buf]).wait()

  lane = lax.iota(jnp.int32, 16)
  w0a = par_v[0, pl.ds(0, 16)]
  w0b = par_v[0, pl.ds(16, 16)]
  w1a = par_v[1, pl.ds(0, 16)]
  w1b = par_v[1, pl.ds(16, 16)]
  brow = par_v[2, pl.ds(0, 16)]
  b0 = brow[0]
  b1 = brow[1]

  def do_row(buf, r):
    drain(buf)
    rr = rows_v.at[buf]

    def acc_body(it, carry):
      a0, a1, a2, a3 = carry
      base = it * UNROLL
      for k in range(UNROLL):
        lo = rr[base + k, pl.ds(0, 16)]
        hi = rr[base + k, pl.ds(16, 16)]
        if k % 2 == 0:
          a0 = a0 + lo
          a1 = a1 + hi
        else:
          a2 = a2 + lo
          a3 = a3 + hi
      return a0, a1, a2, a3

    z = jnp.zeros((16,), jnp.float32)
    a0, a1, a2, a3 = lax.fori_loop(0, L // UNROLL, acc_body, (z, z, z, z))
    mlo = (a0 + a2) * (1.0 / L)
    mhi = (a1 + a3) * (1.0 / L)
    o0 = jnp.sum(mlo * w0a) + jnp.sum(mhi * w0b) + b0
    o1 = jnp.sum(mlo * w1a) + jnp.sum(mhi * w1b) + b1
    vout = jnp.where(lane == 0, o0, jnp.where(lane == 1, o1, 0.0))
    out_v[r] = vout

  for j in range(NBUF):
    fire(j, j)

  def group(g, _):
    for j in range(NBUF):
      r = g * NBUF + j
      do_row(j, r)
      fire(j, r + NBUF)
    return 0

  lax.fori_loop(0, RW // NBUF - 1, group, 0)
  gl = RW // NBUF - 1
  for j in range(NBUF):
    do_row(j, gl * NBUF + j)

  pltpu.sync_copy(out_v, out_hbm.at[pl.ds(w * RW, RW)])


_sc_call = pl.kernel(
    _body,
    out_type=jax.ShapeDtypeStruct((B, 16), jnp.float32),
    mesh=_mesh,
    scratch_types=(
        [pltpu.VMEM((RW * CPR, CH), jnp.int32),
         pltpu.VMEM((NBUF, L, D), jnp.float32),
         pltpu.VMEM((3, D), jnp.float32),
         pltpu.VMEM((RW, 16), jnp.float32)]
        + [pltpu.SemaphoreType.DMA] * NBUF),
    compiler_params=pltpu.CompilerParams(
        needs_layout_passes=False, use_tc_tiling_on_sc=False),
)


@jax.jit
def kernel(x, table, W, b):
  x_chunks = x.reshape(B * L // CH, CH).astype(jnp.int32)
  par = jnp.concatenate(
      [W.T.astype(jnp.float32),
       jnp.pad(b.astype(jnp.float32), (0, D - 2))[None, :]], axis=0)
  out16 = _sc_call(x_chunks, table, par)
  return out16[:, :2]


# speedup vs baseline: 9.5956x; 3.9462x over previous
import functools

import jax
import jax.numpy as jnp
from jax import lax
from jax.experimental import pallas as pl
from jax.experimental.pallas import tpu as pltpu
from jax.experimental.pallas import tpu_sc as plsc

B = 4096
L = 200
D = 32
V = 1000000

NC = 2
NS = 16
NW = NC * NS
RW = B // NW
CH = 40
CPR = L // CH
NBUF = 4
BL = 208

CB = 32768


def _tc_body(wt_ref, tt_ref, m0_ref, m1_ref):
  res = lax.dot_general(
      wt_ref[...], tt_ref[...],
      dimension_numbers=(((1,), (0,)), ((), ())),
      preferred_element_type=jnp.float32)
  m0_ref[...] = res[0, :]
  m1_ref[...] = res[1, :]


_tc_call = pl.pallas_call(
    _tc_body,
    grid=(pl.cdiv(V, CB),),
    in_specs=[pl.BlockSpec((2, D), lambda i: (0, 0)),
              pl.BlockSpec((D, CB), lambda i: (0, i))],
    out_specs=[pl.BlockSpec((CB,), lambda i: (i,)),
               pl.BlockSpec((CB,), lambda i: (i,))],
    out_shape=[jax.ShapeDtypeStruct((V,), jnp.float32),
               jax.ShapeDtypeStruct((V,), jnp.float32)],
)

_mesh = plsc.VectorSubcoreMesh(
    core_axis_name="c", subcore_axis_name="s", num_cores=NC, num_subcores=NS)


def _sc_body(x_hbm, m0_hbm, m1_hbm, b_hbm, out_hbm, idx_v, v0, v1, b_v,
             out_v, *sems):
  c = lax.axis_index("c")
  s = lax.axis_index("s")
  w = s * NC + c

  pltpu.sync_copy(x_hbm.at[pl.ds(w * (RW * CPR), RW * CPR)], idx_v)
  pltpu.sync_copy(b_hbm, b_v)

  def fire(buf, r):
    for vdst, src in ((v0, m0_hbm), (v1, m1_hbm)):
      for h in range(CPR):
        pltpu.async_copy(src.at[idx_v.at[CPR * r + h]],
                         vdst.at[buf, pl.ds(h * CH, CH)], sems[buf])

  def drain(buf):
    for vdst, src in ((v0, m0_hbm), (v1, m1_hbm)):
      pltpu.make_async_copy(src.at[pl.ds(0, L)],
                            vdst.at[buf, pl.ds(0, L)], sems[buf]).wait()

  def zero_pads(buf):
    z = jnp.zeros((16,), jnp.float32)
    for vdst in (v0, v1):
      vdst[buf, pl.ds(192, 16)] = z

  lane = lax.iota(jnp.int32, 16)
  brow = b_v[pl.ds(0, 16)]
  b0 = brow[0]
  b1 = brow[1]

  def do_row(buf, r):
    drain(buf)

    def acc_body(i, carry):
      a0, a1 = carry
      return (a0 + v0[buf, pl.ds(i * 16, 16)],
              a1 + v1[buf, pl.ds(i * 16, 16)])

    z = jnp.zeros((16,), jnp.float32)
    a0, a1 = (z, z)
    for i in range(BL // 16):
      a0, a1 = acc_body(i, (a0, a1))
    o0 = jnp.sum(a0) * (1.0 / L) + b0
    o1 = jnp.sum(a1) * (1.0 / L) + b1
    out_v[r] = jnp.where(lane == 0, o0, jnp.where(lane == 1, o1, 0.0))

  for j in range(NBUF):
    zero_pads(j)
  for j in range(NBUF):
    fire(j, j)

  def group(g, _):
    for j in range(NBUF):
      r = g * NBUF + j
      do_row(j, r)
      fire(j, r + NBUF)
    return 0

  lax.fori_loop(0, RW // NBUF - 1, group, 0)
  gl = RW // NBUF - 1
  for j in range(NBUF):
    do_row(j, gl * NBUF + j)

  pltpu.sync_copy(out_v, out_hbm.at[pl.ds(w * RW, RW)])


_sc_call = pl.kernel(
    _sc_body,
    out_type=jax.ShapeDtypeStruct((B, 16), jnp.float32),
    mesh=_mesh,
    scratch_types=(
        [pltpu.VMEM((RW * CPR, CH), jnp.int32),
         pltpu.VMEM((NBUF, BL), jnp.float32),
         pltpu.VMEM((NBUF, BL), jnp.float32),
         pltpu.VMEM((D,), jnp.float32),
         pltpu.VMEM((RW, 16), jnp.float32)]
        + [pltpu.SemaphoreType.DMA] * NBUF),
    compiler_params=pltpu.CompilerParams(
        needs_layout_passes=False, use_tc_tiling_on_sc=False),
)


@jax.jit
def kernel(x, table, W, b):
  wt = W.T.astype(jnp.float32)
  tt = table.T
  m0, m1 = _tc_call(wt, tt)
  x_chunks = x.reshape(B * L // CH, CH).astype(jnp.int32)
  bpad = jnp.pad(b.astype(jnp.float32), (0, D - 2))
  out16 = _sc_call(x_chunks, m0, m1, bpad)
  return out16[:, :2]


# speedup vs baseline: 11.6586x; 1.2150x over previous
import functools

import jax
import jax.numpy as jnp
from jax import lax
from jax.experimental import pallas as pl
from jax.experimental.pallas import tpu as pltpu
from jax.experimental.pallas import tpu_sc as plsc

B = 4096
L = 200
D = 32
V = 1000000

NC = 2
NS = 16
NW = NC * NS
RW = B // NW
CH = 40
CPR = L // CH
NBUF = 4
BL = 208

CB = 32768


def _tc_body(wt_ref, tt_ref, mpk_ref):
  res = lax.dot_general(
      wt_ref[...], tt_ref[...],
      dimension_numbers=(((1,), (0,)), ((), ())),
      preferred_element_type=jnp.float32)
  lo = lax.bitcast_convert_type(
      res[0, :].astype(jnp.bfloat16), jnp.uint16).astype(jnp.int32)
  hi = lax.bitcast_convert_type(
      res[1, :].astype(jnp.bfloat16), jnp.uint16).astype(jnp.int32)
  mpk_ref[...] = lo | (hi << 16)


_tc_call = pl.pallas_call(
    _tc_body,
    grid=(pl.cdiv(V, CB),),
    in_specs=[pl.BlockSpec((2, D), lambda i: (0, 0)),
              pl.BlockSpec((D, CB), lambda i: (0, i))],
    out_specs=pl.BlockSpec((CB,), lambda i: (i,)),
    out_shape=jax.ShapeDtypeStruct((V,), jnp.int32),
)

_mesh = plsc.VectorSubcoreMesh(
    core_axis_name="c", subcore_axis_name="s", num_cores=NC, num_subcores=NS)


def _sc_body(x_hbm, mpk_hbm, b_hbm, out_hbm, idx_v, vv, b_v, out_v, *sems):
  c = lax.axis_index("c")
  s = lax.axis_index("s")
  w = s * NC + c

  pltpu.sync_copy(x_hbm.at[pl.ds(w * (RW * CPR), RW * CPR)], idx_v)
  pltpu.sync_copy(b_hbm, b_v)

  def fire(buf, r):
    for h in range(CPR):
      pltpu.async_copy(mpk_hbm.at[idx_v.at[CPR * r + h]],
                       vv.at[buf, pl.ds(h * CH, CH)], sems[buf])

  def drain(buf):
    pltpu.make_async_copy(mpk_hbm.at[pl.ds(0, L)],
                          vv.at[buf, pl.ds(0, L)], sems[buf]).wait()

  def zero_pads(buf):
    vv[buf, pl.ds(192, 16)] = jnp.zeros((16,), jnp.int32)

  lane = lax.iota(jnp.int32, 16)
  brow = b_v[pl.ds(0, 16)]
  b0 = brow[0]
  b1 = brow[1]

  def do_row(buf, r):
    drain(buf)
    z = jnp.zeros((16,), jnp.float32)
    a0, a1 = z, z
    for i in range(BL // 16):
      words = vv[buf, pl.ds(i * 16, 16)]
      p0, p1 = plsc.unpack(plsc.bitcast(words, jnp.bfloat16),
                           format=plsc.PackFormat.INTERLEAVED,
                           preferred_element_type=jnp.float32)
      a0 = a0 + p0
      a1 = a1 + p1
    o0 = jnp.sum(a0) * (1.0 / L) + b0
    o1 = jnp.sum(a1) * (1.0 / L) + b1
    out_v[r] = jnp.where(lane == 0, o0, jnp.where(lane == 1, o1, 0.0))

  for j in range(NBUF):
    zero_pads(j)
  for j in range(NBUF):
    fire(j, j)

  def group(g, _):
    for j in range(NBUF):
      r = g * NBUF + j
      do_row(j, r)
      fire(j, r + NBUF)
    return 0

  lax.fori_loop(0, RW // NBUF - 1, group, 0)
  gl = RW // NBUF - 1
  for j in range(NBUF):
    do_row(j, gl * NBUF + j)

  pltpu.sync_copy(out_v, out_hbm.at[pl.ds(w * RW, RW)])


_sc_call = pl.kernel(
    _sc_body,
    out_type=jax.ShapeDtypeStruct((B, 16), jnp.float32),
    mesh=_mesh,
    scratch_types=(
        [pltpu.VMEM((RW * CPR, CH), jnp.int32),
         pltpu.VMEM((NBUF, BL), jnp.int32),
         pltpu.VMEM((D,), jnp.float32),
         pltpu.VMEM((RW, 16), jnp.float32)]
        + [pltpu.SemaphoreType.DMA] * NBUF),
    compiler_params=pltpu.CompilerParams(
        needs_layout_passes=False, use_tc_tiling_on_sc=False),
)


@jax.jit
def kernel(x, table, W, b):
  wt = W.T.astype(jnp.float32)
  tt = table.T
  mpk = _tc_call(wt, tt)
  x_chunks = x.reshape(B * L // CH, CH).astype(jnp.int32)
  bpad = jnp.pad(b.astype(jnp.float32), (0, D - 2))
  out16 = _sc_call(x_chunks, mpk, bpad)
  return out16[:, :2]


# speedup vs baseline: 14.3640x; 1.2321x over previous
import functools

import jax
import jax.numpy as jnp
from jax import lax
from jax.experimental import pallas as pl
from jax.experimental.pallas import tpu as pltpu
from jax.experimental.pallas import tpu_sc as plsc

B = 4096
L = 200
D = 32
V = 1000000

NC = 2
NS = 16
NW = NC * NS
RW = B // NW
CH = 40
CPR = L // CH
NBUF = 4
BL = 208

CB = 32768


def _tc_body(wt_ref, tt_ref, mpk_ref):
  res = lax.dot_general(
      wt_ref[...], tt_ref[...],
      dimension_numbers=(((1,), (0,)), ((), ())),
      preferred_element_type=jnp.float32)
  lo = lax.bitcast_convert_type(
      res[0, :].astype(jnp.bfloat16), jnp.uint16).astype(jnp.int32)
  hi = lax.bitcast_convert_type(
      res[1, :].astype(jnp.bfloat16), jnp.uint16).astype(jnp.int32)
  mpk_ref[...] = lo | (hi << 16)


_tc_call = pl.pallas_call(
    _tc_body,
    grid=(pl.cdiv(V, CB),),
    in_specs=[pl.BlockSpec((2, D), lambda i: (0, 0)),
              pl.BlockSpec((D, CB), lambda i: (0, i))],
    out_specs=pl.BlockSpec((CB,), lambda i: (i,)),
    out_shape=jax.ShapeDtypeStruct((V,), jnp.int32),
)

_mesh = plsc.VectorSubcoreMesh(
    core_axis_name="c", subcore_axis_name="s", num_cores=NC, num_subcores=NS)


def _sc_body(x_hbm, mpk_hbm, b_hbm, out_hbm, idx_v, vv, b_v, out_v, mpk_sh,
             *sems):
  c = lax.axis_index("c")
  s = lax.axis_index("s")
  w = s * NC + c

  pltpu.sync_copy(x_hbm.at[pl.ds(w * (RW * CPR), RW * CPR)], idx_v)
  pltpu.sync_copy(b_hbm, b_v)

  SH = 62496
  pltpu.sync_copy(mpk_hbm.at[pl.ds(s * SH, SH)],
                  mpk_sh.at[pl.ds(s * SH, SH)])
  @pl.when(s == NS - 1)
  def _():
    pltpu.sync_copy(mpk_hbm.at[pl.ds(NS * SH, V - NS * SH)],
                    mpk_sh.at[pl.ds(NS * SH, V - NS * SH)])
  plsc.subcore_barrier()

  def fire(buf, r):
    for h in range(CPR):
      pltpu.async_copy(mpk_sh.at[idx_v.at[CPR * r + h]],
                       vv.at[buf, pl.ds(h * CH, CH)], sems[buf])

  def drain(buf):
    pltpu.make_async_copy(mpk_hbm.at[pl.ds(0, L)],
                          vv.at[buf, pl.ds(0, L)], sems[buf]).wait()

  def zero_pads(buf):
    vv[buf, pl.ds(192, 16)] = jnp.zeros((16,), jnp.int32)

  lane = lax.iota(jnp.int32, 16)
  brow = b_v[pl.ds(0, 16)]
  b0 = brow[0]
  b1 = brow[1]

  def do_row(buf, r):
    drain(buf)
    z = jnp.zeros((16,), jnp.float32)
    a0, a1 = z, z
    for i in range(BL // 16):
      words = vv[buf, pl.ds(i * 16, 16)]
      p0, p1 = plsc.unpack(plsc.bitcast(words, jnp.bfloat16),
                           format=plsc.PackFormat.INTERLEAVED,
                           preferred_element_type=jnp.float32)
      a0 = a0 + p0
      a1 = a1 + p1
    o0 = jnp.sum(a0) * (1.0 / L) + b0
    o1 = jnp.sum(a1) * (1.0 / L) + b1
    out_v[r] = jnp.where(lane == 0, o0, jnp.where(lane == 1, o1, 0.0))

  for j in range(NBUF):
    zero_pads(j)
  for j in range(NBUF):
    fire(j, j)

  def group(g, _):
    for j in range(NBUF):
      r = g * NBUF + j
      do_row(j, r)
      fire(j, r + NBUF)
    return 0

  lax.fori_loop(0, RW // NBUF - 1, group, 0)
  gl = RW // NBUF - 1
  for j in range(NBUF):
    do_row(j, gl * NBUF + j)

  pltpu.sync_copy(out_v, out_hbm.at[pl.ds(w * RW, RW)])


_sc_call = pl.kernel(
    _sc_body,
    out_type=jax.ShapeDtypeStruct((B, 16), jnp.float32),
    mesh=_mesh,
    scratch_types=(
        [pltpu.VMEM((RW * CPR, CH), jnp.int32),
         pltpu.VMEM((NBUF, BL), jnp.int32),
         pltpu.VMEM((D,), jnp.float32),
         pltpu.VMEM((RW, 16), jnp.float32),
         pltpu.VMEM_SHARED((V,), jnp.int32)]
        + [pltpu.SemaphoreType.DMA] * NBUF),
    compiler_params=pltpu.CompilerParams(
        needs_layout_passes=False, use_tc_tiling_on_sc=False),
)


@jax.jit
def kernel(x, table, W, b):
  wt = W.T.astype(jnp.float32)
  tt = table.T
  mpk = _tc_call(wt, tt)
  x_chunks = x.reshape(B * L // CH, CH).astype(jnp.int32)
  bpad = jnp.pad(b.astype(jnp.float32), (0, D - 2))
  out16 = _sc_call(x_chunks, mpk, bpad)
  return out16[:, :2]


# speedup vs baseline: 15.1694x; 1.0561x over previous
import functools

import jax
import jax.numpy as jnp
from jax import lax
from jax.experimental import pallas as pl
from jax.experimental.pallas import tpu as pltpu
from jax.experimental.pallas import tpu_sc as plsc

B = 4096
L = 200
D = 32
V = 1000000

NC = 2
NS = 16
NW = NC * NS
RW = B // NW
CH = 40
CPR = L // CH
NBUF = 4
BL = 208

CB = 65536


def _tc_body(wt_ref, tt_ref, mpk_ref):
  res = lax.dot_general(
      wt_ref[...], tt_ref[...],
      dimension_numbers=(((1,), (0,)), ((), ())),
      preferred_element_type=jnp.float32)
  lo = lax.bitcast_convert_type(
      res[0, :].astype(jnp.bfloat16), jnp.uint16).astype(jnp.int32)
  hi = lax.bitcast_convert_type(
      res[1, :].astype(jnp.bfloat16), jnp.uint16).astype(jnp.int32)
  mpk_ref[...] = lo | (hi << 16)


_tc_call = pl.pallas_call(
    _tc_body,
    grid=(pl.cdiv(V, CB),),
    in_specs=[pl.BlockSpec((2, D), lambda i: (0, 0)),
              pl.BlockSpec((D, CB), lambda i: (0, i))],
    out_specs=pl.BlockSpec((CB,), lambda i: (i,)),
    out_shape=jax.ShapeDtypeStruct((V,), jnp.int32),
    compiler_params=pltpu.CompilerParams(
        dimension_semantics=("arbitrary",),
        vmem_limit_bytes=100 * 1024 * 1024),
)

_mesh = plsc.VectorSubcoreMesh(
    core_axis_name="c", subcore_axis_name="s", num_cores=NC, num_subcores=NS)


def _sc_body(x_hbm, mpk_hbm, b_hbm, out_hbm, idx_v, vv, b_v, out_v, mpk_sh,
             *sems):
  c = lax.axis_index("c")
  s = lax.axis_index("s")
  w = s * NC + c

  pltpu.sync_copy(x_hbm.at[pl.ds(w * (RW * CPR), RW * CPR)], idx_v)
  pltpu.sync_copy(b_hbm, b_v)

  SH = 62496
  pltpu.sync_copy(mpk_hbm.at[pl.ds(s * SH, SH)],
                  mpk_sh.at[pl.ds(s * SH, SH)])
  @pl.when(s == NS - 1)
  def _():
    pltpu.sync_copy(mpk_hbm.at[pl.ds(NS * SH, V - NS * SH)],
                    mpk_sh.at[pl.ds(NS * SH, V - NS * SH)])
  plsc.subcore_barrier()

  def fire(buf, r):
    for h in range(CPR):
      pltpu.async_copy(mpk_sh.at[idx_v.at[CPR * r + h]],
                       vv.at[buf, pl.ds(h * CH, CH)], sems[buf])

  def drain(buf):
    pltpu.make_async_copy(mpk_hbm.at[pl.ds(0, L)],
                          vv.at[buf, pl.ds(0, L)], sems[buf]).wait()

  def zero_pads(buf):
    vv[buf, pl.ds(192, 16)] = jnp.zeros((16,), jnp.int32)

  lane = lax.iota(jnp.int32, 16)
  brow = b_v[pl.ds(0, 16)]
  b0 = brow[0]
  b1 = brow[1]

  def do_row(buf, r):
    drain(buf)
    z = jnp.zeros((16,), jnp.float32)
    a0, a1 = z, z
    for i in range(BL // 16):
      words = vv[buf, pl.ds(i * 16, 16)]
      p0, p1 = plsc.unpack(plsc.bitcast(words, jnp.bfloat16),
                           format=plsc.PackFormat.INTERLEAVED,
                           preferred_element_type=jnp.float32)
      a0 = a0 + p0
      a1 = a1 + p1
    o0 = jnp.sum(a0) * (1.0 / L) + b0
    o1 = jnp.sum(a1) * (1.0 / L) + b1
    out_v[r] = jnp.where(lane == 0, o0, jnp.where(lane == 1, o1, 0.0))

  for j in range(NBUF):
    zero_pads(j)
  for j in range(NBUF):
    fire(j, j)

  def group(g, _):
    for j in range(NBUF):
      r = g * NBUF + j
      do_row(j, r)
      fire(j, r + NBUF)
    return 0

  lax.fori_loop(0, RW // NBUF - 1, group, 0)
  gl = RW // NBUF - 1
  for j in range(NBUF):
    do_row(j, gl * NBUF + j)

  pltpu.sync_copy(out_v, out_hbm.at[pl.ds(w * RW, RW)])


_sc_call = pl.kernel(
    _sc_body,
    out_type=jax.ShapeDtypeStruct((B, 16), jnp.float32),
    mesh=_mesh,
    scratch_types=(
        [pltpu.VMEM((RW * CPR, CH), jnp.int32),
         pltpu.VMEM((NBUF, BL), jnp.int32),
         pltpu.VMEM((D,), jnp.float32),
         pltpu.VMEM((RW, 16), jnp.float32),
         pltpu.VMEM_SHARED((V,), jnp.int32)]
        + [pltpu.SemaphoreType.DMA] * NBUF),
    compiler_params=pltpu.CompilerParams(
        needs_layout_passes=False, use_tc_tiling_on_sc=False),
)


@jax.jit
def kernel(x, table, W, b):
  wt = W.T.astype(jnp.float32)
  tt = table.T
  mpk = _tc_call(wt, tt)
  x_chunks = x.reshape(B * L // CH, CH).astype(jnp.int32)
  bpad = jnp.pad(b.astype(jnp.float32), (0, D - 2))
  out16 = _sc_call(x_chunks, mpk, bpad)
  return out16[:, :2]


# speedup vs baseline: 16.0550x; 1.0584x over previous
import functools

import jax
import jax.numpy as jnp
from jax import lax
from jax.experimental import pallas as pl
from jax.experimental.pallas import tpu as pltpu
from jax.experimental.pallas import tpu_sc as plsc

B = 4096
L = 200
D = 32
V = 1000000

NC = 2
NS = 16
NW = NC * NS
RW = B // NW
CH = 40
CPR = L // CH
NBUF = 4
BL = 208

CB = 65536


def _tc_body(wt_ref, tt_ref, mpk_ref):
  res = lax.dot_general(
      wt_ref[...], tt_ref[...],
      dimension_numbers=(((1,), (0,)), ((), ())),
      preferred_element_type=jnp.float32)
  lo = lax.bitcast_convert_type(
      res[0, :].astype(jnp.bfloat16), jnp.uint16).astype(jnp.int32)
  hi = lax.bitcast_convert_type(
      res[1, :].astype(jnp.bfloat16), jnp.uint16).astype(jnp.int32)
  mpk_ref[...] = lo | (hi << 16)


_tc_call = pl.pallas_call(
    _tc_body,
    grid=(pl.cdiv(V, CB),),
    in_specs=[pl.BlockSpec((2, D), lambda i: (0, 0)),
              pl.BlockSpec((D, CB), lambda i: (0, i))],
    out_specs=pl.BlockSpec((CB,), lambda i: (i,)),
    out_shape=jax.ShapeDtypeStruct((V,), jnp.int32),
    compiler_params=pltpu.CompilerParams(
        dimension_semantics=("arbitrary",),
        vmem_limit_bytes=100 * 1024 * 1024),
)

_mesh = plsc.VectorSubcoreMesh(
    core_axis_name="c", subcore_axis_name="s", num_cores=NC, num_subcores=NS)


def _prep_body(xt_hbm, out_hbm, xin_v, idxT_v):
  c = lax.axis_index("c")
  s = lax.axis_index("s")
  w = s * NC + c
  pltpu.sync_copy(xt_hbm.at[:, pl.ds(w * RW, RW)], xin_v)
  lane5 = lax.iota(jnp.int32, 16) * CPR
  for t in range(L):
    col = jnp.full((16,), t % CH, jnp.int32)
    for k in range(RW // 16):
      vals = xin_v[t, pl.ds(k * 16, 16)]
      rows = lane5 + (k * 16 * CPR + t // CH)
      plsc.store_scatter(idxT_v, [rows, col], vals)
  pltpu.sync_copy(idxT_v, out_hbm.at[pl.ds(w * (RW * CPR), RW * CPR)])


_prep_call = pl.kernel(
    _prep_body,
    out_type=jax.ShapeDtypeStruct((B * L // CH, CH), jnp.int32),
    mesh=_mesh,
    scratch_types=[pltpu.VMEM((L, RW), jnp.int32),
                   pltpu.VMEM((RW * CPR, CH), jnp.int32)],
    compiler_params=pltpu.CompilerParams(
        needs_layout_passes=False, use_tc_tiling_on_sc=False),
)


def _sc_body(x_hbm, mpk_hbm, b_hbm, out_hbm, idx_v, vv, b_v, out_v, mpk_sh,
             *sems):
  c = lax.axis_index("c")
  s = lax.axis_index("s")
  w = s * NC + c

  pltpu.sync_copy(x_hbm.at[pl.ds(w * (RW * CPR), RW * CPR)], idx_v)
  pltpu.sync_copy(b_hbm, b_v)

  SH = 62496
  pltpu.sync_copy(mpk_hbm.at[pl.ds(s * SH, SH)],
                  mpk_sh.at[pl.ds(s * SH, SH)])
  @pl.when(s == NS - 1)
  def _():
    pltpu.sync_copy(mpk_hbm.at[pl.ds(NS * SH, V - NS * SH)],
                    mpk_sh.at[pl.ds(NS * SH, V - NS * SH)])
  plsc.subcore_barrier()

  def fire(buf, r):
    for h in range(CPR):
      pltpu.async_copy(mpk_sh.at[idx_v.at[CPR * r + h]],
                       vv.at[buf, pl.ds(h * CH, CH)], sems[buf])

  def drain(buf):
    pltpu.make_async_copy(mpk_hbm.at[pl.ds(0, L)],
                          vv.at[buf, pl.ds(0, L)], sems[buf]).wait()

  def zero_pads(buf):
    vv[buf, pl.ds(192, 16)] = jnp.zeros((16,), jnp.int32)

  lane = lax.iota(jnp.int32, 16)
  brow = b_v[pl.ds(0, 16)]
  b0 = brow[0]
  b1 = brow[1]

  def do_row(buf, r):
    drain(buf)
    z = jnp.zeros((16,), jnp.float32)
    a0, a1 = z, z
    for i in range(BL // 16):
      words = vv[buf, pl.ds(i * 16, 16)]
      p0, p1 = plsc.unpack(plsc.bitcast(words, jnp.bfloat16),
                           format=plsc.PackFormat.INTERLEAVED,
                           preferred_element_type=jnp.float32)
      a0 = a0 + p0
      a1 = a1 + p1
    o0 = jnp.sum(a0) * (1.0 / L) + b0
    o1 = jnp.sum(a1) * (1.0 / L) + b1
    out_v[r] = jnp.where(lane == 0, o0, jnp.where(lane == 1, o1, 0.0))

  for j in range(NBUF):
    zero_pads(j)
  for j in range(NBUF):
    fire(j, j)

  def group(g, _):
    for j in range(NBUF):
      r = g * NBUF + j
      do_row(j, r)
      fire(j, r + NBUF)
    return 0

  lax.fori_loop(0, RW // NBUF - 1, group, 0)
  gl = RW // NBUF - 1
  for j in range(NBUF):
    do_row(j, gl * NBUF + j)

  pltpu.sync_copy(out_v, out_hbm.at[pl.ds(w * RW, RW)])


_sc_call = pl.kernel(
    _sc_body,
    out_type=jax.ShapeDtypeStruct((B, 16), jnp.float32),
    mesh=_mesh,
    scratch_types=(
        [pltpu.VMEM((RW * CPR, CH), jnp.int32),
         pltpu.VMEM((NBUF, BL), jnp.int32),
         pltpu.VMEM((D,), jnp.float32),
         pltpu.VMEM((RW, 16), jnp.float32),
         pltpu.VMEM_SHARED((V,), jnp.int32)]
        + [pltpu.SemaphoreType.DMA] * NBUF),
    compiler_params=pltpu.CompilerParams(
        needs_layout_passes=False, use_tc_tiling_on_sc=False),
)


@jax.jit
def kernel(x, table, W, b):
  wt = W.T.astype(jnp.float32)
  tt = table.T
  mpk = _tc_call(wt, tt)
  x_chunks = _prep_call(x.astype(jnp.int32).T)
  bpad = jnp.pad(b.astype(jnp.float32), (0, D - 2))
  out16 = _sc_call(x_chunks, mpk, bpad)
  return out16[:, :2]
